# baseline (speedup 1.0000x reference)
import jax
import jax.numpy as jnp
from jax.experimental import pallas as pl
from jax.experimental.pallas import tpu as pltpu

N = 10000
D = 128
BLOCK = 1000


def _mlp3_kernel(x_ref, w0_ref, w1_ref, w2_ref, b_ref, out_ref):
    x = x_ref[...]
    h = jnp.dot(x, w0_ref[...], preferred_element_type=jnp.float32)
    h = jnp.maximum(h + b_ref[0, :], 0.0)
    h = jnp.dot(h, w1_ref[...], preferred_element_type=jnp.float32)
    h = jnp.maximum(h + b_ref[1, :], 0.0)
    h = jnp.dot(h, w2_ref[...], preferred_element_type=jnp.float32)
    out_ref[...] = h + b_ref[2, :]


def kernel(x, edge_index, edge_weight, W0, b0, W1, b1, W2, b2):
    wt0 = W0.T
    wt1 = W1.T
    wt2 = W2.T
    b = jnp.stack([b0, b1, b2])

    grid = (N // BLOCK,)
    full = pl.BlockSpec((D, D), lambda i: (0, 0))
    out = pl.pallas_call(
        _mlp3_kernel,
        grid=grid,
        in_specs=[
            pl.BlockSpec((BLOCK, D), lambda i: (i, 0)),
            full,
            full,
            full,
            pl.BlockSpec((3, D), lambda i: (0, 0)),
        ],
        out_specs=pl.BlockSpec((BLOCK, D), lambda i: (i, 0)),
        out_shape=jax.ShapeDtypeStruct((N, D), jnp.float32),
        compiler_params=pltpu.CompilerParams(
            dimension_semantics=("parallel",),
        ),
    )(x, wt0, wt1, wt2, b)
    return out


# speedup vs baseline: 2.1869x; 2.1869x over previous
import jax
import jax.numpy as jnp
from jax.experimental import pallas as pl
from jax.experimental.pallas import tpu as pltpu

N = 10000
D = 128
BLOCK = 2000

_DN = (((1,), (1,)), ((), ()))


def _mlp3_kernel(x_ref, w0_ref, w1_ref, w2_ref, b0_ref, b1_ref, b2_ref,
                 out_ref):
    x = x_ref[...]
    h = jax.lax.dot_general(x, w0_ref[...], _DN,
                            preferred_element_type=jnp.float32)
    h = jnp.maximum(h + b0_ref[...], 0.0)
    h = jax.lax.dot_general(h, w1_ref[...], _DN,
                            preferred_element_type=jnp.float32)
    h = jnp.maximum(h + b1_ref[...], 0.0)
    h = jax.lax.dot_general(h, w2_ref[...], _DN,
                            preferred_element_type=jnp.float32)
    out_ref[...] = h + b2_ref[...]


def kernel(x, edge_index, edge_weight, W0, b0, W1, b1, W2, b2):
    grid = (N // BLOCK,)
    full = pl.BlockSpec((D, D), lambda i: (0, 0))
    brow = pl.BlockSpec((1, D), lambda i: (0, 0))
    out = pl.pallas_call(
        _mlp3_kernel,
        grid=grid,
        in_specs=[
            pl.BlockSpec((BLOCK, D), lambda i: (i, 0)),
            full, full, full,
            brow, brow, brow,
        ],
        out_specs=pl.BlockSpec((BLOCK, D), lambda i: (i, 0)),
        out_shape=jax.ShapeDtypeStruct((N, D), jnp.float32),
        compiler_params=pltpu.CompilerParams(
            dimension_semantics=("parallel",),
        ),
    )(x, W0, W1, W2,
      b0.reshape(1, D), b1.reshape(1, D), b2.reshape(1, D))
    return out
